# baseline (speedup 1.0000x reference)
import functools

import jax
import jax.numpy as jnp
from jax import lax
from jax.experimental import pallas as pl
from jax.experimental.pallas import tpu as pltpu
from jax.experimental.pallas import tpu_sc as plsc

N = 10000
E = 320000
D = 128

NC = 2
NS = 16
L = 16

EDGES_PER_CORE = E // NC
EDGES_PER_TILE = E // (NC * NS)
CHUNK = 128
NCHUNKS = -(-EDGES_PER_TILE // CHUNK)
EPT_PAD = NCHUNKS * CHUNK
ROWS_PER_TILE = 624

_TC_BLK = 1000


def _tc_pre_body(h_ref, wr_ref, wt_ref, b_ref, y_ref, z_ref):
    h = h_ref[...]
    y_ref[...] = jnp.dot(h, wr_ref[...], preferred_element_type=jnp.float32)
    z_ref[...] = jnp.dot(h, wt_ref[...], preferred_element_type=jnp.float32) + b_ref[...]


def _tc_pre(h, wr, wt, b):
    grid = (N // _TC_BLK,)
    return pl.pallas_call(
        _tc_pre_body,
        grid=grid,
        in_specs=[
            pl.BlockSpec((_TC_BLK, D), lambda i: (i, 0)),
            pl.BlockSpec((D, D), lambda i: (0, 0)),
            pl.BlockSpec((D, D), lambda i: (0, 0)),
            pl.BlockSpec((1, D), lambda i: (0, 0)),
        ],
        out_specs=[
            pl.BlockSpec((_TC_BLK, D), lambda i: (i, 0)),
            pl.BlockSpec((_TC_BLK, D), lambda i: (i, 0)),
        ],
        out_shape=[
            jax.ShapeDtypeStruct((N, D), jnp.float32),
            jax.ShapeDtypeStruct((N, D), jnp.float32),
        ],
    )(h, wr, wt, b.reshape(1, D))


def _tc_mid_body(p_ref, z_ref, wr_ref, wt_ref, b_ref, y_ref, z2_ref):
    h = jax.nn.relu(p_ref[0] + p_ref[1] + z_ref[...])
    y_ref[...] = jnp.dot(h, wr_ref[...], preferred_element_type=jnp.float32)
    z2_ref[...] = jnp.dot(h, wt_ref[...], preferred_element_type=jnp.float32) + b_ref[...]


def _tc_mid(p, z, wr, wt, b):
    grid = (N // _TC_BLK,)
    return pl.pallas_call(
        _tc_mid_body,
        grid=grid,
        in_specs=[
            pl.BlockSpec((2, _TC_BLK, D), lambda i: (0, i, 0)),
            pl.BlockSpec((_TC_BLK, D), lambda i: (i, 0)),
            pl.BlockSpec((D, D), lambda i: (0, 0)),
            pl.BlockSpec((D, D), lambda i: (0, 0)),
            pl.BlockSpec((1, D), lambda i: (0, 0)),
        ],
        out_specs=[
            pl.BlockSpec((_TC_BLK, D), lambda i: (i, 0)),
            pl.BlockSpec((_TC_BLK, D), lambda i: (i, 0)),
        ],
        out_shape=[
            jax.ShapeDtypeStruct((N, D), jnp.float32),
            jax.ShapeDtypeStruct((N, D), jnp.float32),
        ],
    )(p, z, wr, wt, b.reshape(1, D))


def _tc_post_body(p_ref, z_ref, o_ref):
    o_ref[...] = jax.nn.relu(p_ref[0] + p_ref[1] + z_ref[...])


def _tc_post(p, z):
    grid = (N // _TC_BLK,)
    return pl.pallas_call(
        _tc_post_body,
        grid=grid,
        in_specs=[
            pl.BlockSpec((2, _TC_BLK, D), lambda i: (0, i, 0)),
            pl.BlockSpec((_TC_BLK, D), lambda i: (i, 0)),
        ],
        out_specs=pl.BlockSpec((_TC_BLK, D), lambda i: (i, 0)),
        out_shape=jax.ShapeDtypeStruct((N, D), jnp.float32),
    )(p, z)


def _sc_agg_body(y_hbm, idx_hbm, w_hbm, out_hbm,
                 ibuf0, ibuf1, w_v, rows0_v, rows1_v, acc_sh,
                 semg0, semg1):
    c = lax.axis_index("c")
    s = lax.axis_index("s")
    tid = c * NS + s

    def zb_body(i, _):
        for g in range(D // L):
            rows0_v[i, pl.ds(g * L, L)] = jnp.zeros((L,), jnp.float32)
        return 0
    lax.fori_loop(0, CHUNK, zb_body, 0)

    r0 = s * ROWS_PER_TILE
    for k in range(4):
        pltpu.sync_copy(rows0_v, acc_sh.at[pl.ds(r0 + k * 128, 128)])

    @pl.when(s == NS - 1)
    def _():
        pltpu.sync_copy(rows0_v, acc_sh.at[pl.ds(r0 + 512, 128)])

    @pl.when(s < NS - 1)
    def _():
        pltpu.sync_copy(rows0_v.at[pl.ds(0, 112)], acc_sh.at[pl.ds(r0 + 512, 112)])

    plsc.subcore_barrier()

    pltpu.sync_copy(w_hbm.at[tid], w_v)

    def load_idx(k, ibuf):
        pltpu.sync_copy(idx_hbm.at[tid, k], ibuf)

    def start_gather(ibuf, buf, sem):
        return pltpu.async_copy(y_hbm.at[ibuf.at[0]], buf, sem)

    def wait_gather(buf, sem):
        pltpu.make_async_copy(y_hbm.at[ibuf0.at[0]], buf, sem).wait()

    def process(k, ibuf, buf):
        def grp_body(gi, _):
            w16 = w_v[k, pl.ds(gi * L, L)]
            for i in range(L):
                wb = jnp.full((L,), w16[i], dtype=jnp.float32)
                r = gi * L + i
                for g in range(D // L):
                    buf[r, pl.ds(g * L, L)] = buf[r, pl.ds(g * L, L)] * wb
            return 0
        lax.fori_loop(0, CHUNK // L, grp_body, 0)
        pltpu.sync_copy(buf, acc_sh.at[ibuf.at[1]], add=True)

    last = NCHUNKS - 1
    load_idx(0, ibuf0)
    load_idx(1, ibuf1)
    start_gather(ibuf0, rows0_v, semg0)
    start_gather(ibuf1, rows1_v, semg1)

    def slot(k, ibuf, buf, sem):
        wait_gather(buf, sem)
        process(k, ibuf, buf)
        load_idx(jnp.minimum(k + 2, last), ibuf)
        start_gather(ibuf, buf, sem)

    def pipe_body(g, _):
        k0 = 2 * g
        slot(k0, ibuf0, rows0_v, semg0)
        slot(k0 + 1, ibuf1, rows1_v, semg1)
        return 0

    lax.fori_loop(0, (NCHUNKS - 1) // 2, pipe_body, 0)

    wait_gather(rows0_v, semg0)
    process(last, ibuf0, rows0_v)
    wait_gather(rows1_v, semg1)

    plsc.subcore_barrier()

    ob = c * N + r0

    @pl.when(s == NS - 1)
    def _():
        pltpu.sync_copy(acc_sh.at[pl.ds(r0, 640)], out_hbm.at[pl.ds(ob, 640)])

    @pl.when(s < NS - 1)
    def _():
        pltpu.sync_copy(acc_sh.at[pl.ds(r0, ROWS_PER_TILE)],
                        out_hbm.at[pl.ds(ob, ROWS_PER_TILE)])


@functools.partial(jax.jit, static_argnames=())
def _sc_agg(y, idx_p, w_p):
    mesh = plsc.VectorSubcoreMesh(core_axis_name="c", subcore_axis_name="s",
                                  num_cores=NC, num_subcores=NS)
    k = pl.kernel(
        _sc_agg_body,
        out_type=jax.ShapeDtypeStruct((2 * N, D), jnp.float32),
        mesh=mesh,
        scratch_types=[
            pltpu.VMEM((2, CHUNK), jnp.int32),
            pltpu.VMEM((2, CHUNK), jnp.int32),
            pltpu.VMEM((NCHUNKS, CHUNK), jnp.float32),
            pltpu.VMEM((CHUNK, D), jnp.float32),
            pltpu.VMEM((CHUNK, D), jnp.float32),
            pltpu.VMEM_SHARED((N, D), jnp.float32),
            pltpu.SemaphoreType.DMA,
            pltpu.SemaphoreType.DMA,
        ],
    )
    return k(y, idx_p, w_p).reshape(2, N, D)


def _pad_edges(src, dst, w):
    pad = EPT_PAD - EDGES_PER_TILE
    srcp = jnp.concatenate(
        [src.reshape(NC * NS, EDGES_PER_TILE),
         jnp.zeros((NC * NS, pad), jnp.int32)], axis=1)
    dstp = jnp.concatenate(
        [dst.reshape(NC * NS, EDGES_PER_TILE),
         jnp.zeros((NC * NS, pad), jnp.int32)], axis=1)
    wp = jnp.concatenate(
        [w.reshape(NC * NS, EDGES_PER_TILE),
         jnp.zeros((NC * NS, pad), jnp.float32)], axis=1)
    idx_p = jnp.stack(
        [srcp.reshape(NC * NS, NCHUNKS, CHUNK),
         dstp.reshape(NC * NS, NCHUNKS, CHUNK)], axis=2)
    return idx_p, wp.reshape(NC * NS, NCHUNKS, CHUNK)


def kernel(x, edge_index, edge_attr, batch,
           W_rel0, b_rel0, W_root0,
           W_rel1, b_rel1, W_root1,
           W_rel2, b_rel2, W_root2):
    src = edge_index[0]
    dst = edge_index[1]
    idx_p, w_p = _pad_edges(src, dst, edge_attr)

    y0, z0 = _tc_pre(x, W_rel0, W_root0, b_rel0)
    p0 = _sc_agg(y0, idx_p, w_p)
    y1, z1 = _tc_mid(p0, z0, W_rel1, W_root1, b_rel1)
    p1 = _sc_agg(y1, idx_p, w_p)
    y2, z2 = _tc_mid(p1, z1, W_rel2, W_root2, b_rel2)
    p2 = _sc_agg(y2, idx_p, w_p)
    return _tc_post(p2, z2)
